# baseline (speedup 1.0000x reference)
import jax
import jax.numpy as jnp
from jax.experimental import pallas as pl
from jax.experimental.pallas import tpu as pltpu


def _cc_kernel(cam_ref, wtab_ref, btab_ref, tex_ref, out_ref):
    g = pl.program_id(0)
    b = g // 3
    c = g % 3
    idx = cam_ref[b]
    w = wtab_ref[idx, c]
    bb = btab_ref[idx, c]
    out_ref[...] = tex_ref[...] * w + bb


def kernel(texture, cam, weight, bias):
    B, C, H, W = texture.shape
    n_cam = weight.shape[0] + 1
    wtab = jnp.concatenate(
        [jnp.ones((1, C), texture.dtype), weight.reshape(n_cam - 1, C)], axis=0
    )
    btab = jnp.concatenate(
        [jnp.zeros((1, C), texture.dtype), bias.reshape(n_cam - 1, C)], axis=0
    )
    cam32 = cam.astype(jnp.int32)
    tex2 = texture.reshape(B * C, H, W)
    out = pl.pallas_call(
        _cc_kernel,
        grid=(B * C,),
        in_specs=[
            pl.BlockSpec(memory_space=pltpu.SMEM),
            pl.BlockSpec(memory_space=pltpu.SMEM),
            pl.BlockSpec(memory_space=pltpu.SMEM),
            pl.BlockSpec((1, H, W), lambda g: (g, 0, 0)),
        ],
        out_specs=pl.BlockSpec((1, H, W), lambda g: (g, 0, 0)),
        out_shape=jax.ShapeDtypeStruct((B * C, H, W), texture.dtype),
    )(cam32, wtab, btab, tex2)
    return out.reshape(B, C, H, W)


# speedup vs baseline: 1.2400x; 1.2400x over previous
import jax
import jax.numpy as jnp
from jax.experimental import pallas as pl
from jax.experimental.pallas import tpu as pltpu


def _cc_kernel(cam_ref, wtab_ref, btab_ref, tex_ref, out_ref):
    b = pl.program_id(0)
    idx = cam_ref[b]
    for c in range(3):
        w = wtab_ref[idx, c]
        bb = btab_ref[idx, c]
        out_ref[0, c] = tex_ref[0, c] * w + bb


def kernel(texture, cam, weight, bias):
    B, C, H, W = texture.shape
    n_cam = weight.shape[0] + 1
    wtab = jnp.concatenate(
        [jnp.ones((1, C), texture.dtype), weight.reshape(n_cam - 1, C)], axis=0
    )
    btab = jnp.concatenate(
        [jnp.zeros((1, C), texture.dtype), bias.reshape(n_cam - 1, C)], axis=0
    )
    cam32 = cam.astype(jnp.int32)
    HS = 2
    return pl.pallas_call(
        _cc_kernel,
        grid=(B, HS),
        in_specs=[
            pl.BlockSpec(memory_space=pltpu.SMEM),
            pl.BlockSpec(memory_space=pltpu.SMEM),
            pl.BlockSpec(memory_space=pltpu.SMEM),
            pl.BlockSpec((1, C, H // HS, W), lambda b, h: (b, 0, h, 0)),
        ],
        out_specs=pl.BlockSpec((1, C, H // HS, W), lambda b, h: (b, 0, h, 0)),
        out_shape=jax.ShapeDtypeStruct((B, C, H, W), texture.dtype),
    )(cam32, wtab, btab, texture)


# speedup vs baseline: 1.5515x; 1.2512x over previous
import jax
import jax.numpy as jnp
from jax.experimental import pallas as pl
from jax.experimental.pallas import tpu as pltpu


def _cc_kernel(cam_ref, wtab_ref, btab_ref, tex_ref, out_ref):
    g = pl.program_id(0)
    nb = tex_ref.shape[0]
    for i in range(nb):
        idx = cam_ref[g * nb + i]
        for c in range(3):
            w = wtab_ref[idx, c]
            bb = btab_ref[idx, c]
            out_ref[i, c] = tex_ref[i, c] * w + bb


def kernel(texture, cam, weight, bias):
    B, C, H, W = texture.shape
    n_cam = weight.shape[0] + 1
    wtab = jnp.concatenate(
        [jnp.ones((1, C), texture.dtype), weight.reshape(n_cam - 1, C)], axis=0
    )
    btab = jnp.concatenate(
        [jnp.zeros((1, C), texture.dtype), bias.reshape(n_cam - 1, C)], axis=0
    )
    cam32 = cam.astype(jnp.int32)
    NB = 2
    return pl.pallas_call(
        _cc_kernel,
        grid=(B // NB,),
        in_specs=[
            pl.BlockSpec(memory_space=pltpu.SMEM),
            pl.BlockSpec(memory_space=pltpu.SMEM),
            pl.BlockSpec(memory_space=pltpu.SMEM),
            pl.BlockSpec((NB, C, H, W), lambda g: (g, 0, 0, 0)),
        ],
        out_specs=pl.BlockSpec((NB, C, H, W), lambda g: (g, 0, 0, 0)),
        out_shape=jax.ShapeDtypeStruct((B, C, H, W), texture.dtype),
    )(cam32, wtab, btab, texture)
